# baseline (speedup 1.0000x reference)
import functools

import jax
import jax.numpy as jnp
from jax import lax
from jax.experimental import pallas as pl
from jax.experimental.pallas import tpu as pltpu
from jax.experimental.pallas import tpu_sc as plsc

_ACTION_DIM = 1024
_ACTION = (7, 42, 123)
_LANES = 16
_NUM_WORKERS = 32
_ROWS_PER_BUF = 64


def _fill_body(out_hbm, buf, sem):
    wid = lax.axis_index("s") * 2 + lax.axis_index("c")
    lane = lax.iota(jnp.int32, _LANES)

    def _fill_row(r, carry):
        for g in range(_ACTION_DIM // _LANES):
            base_col = g * _LANES
            v = jnp.zeros((_LANES,), jnp.float32)
            for a in _ACTION:
                if base_col <= a < base_col + _LANES:
                    v = jnp.where(lane == (a - base_col), 1.0, v)
            buf[r, pl.ds(base_col, _LANES)] = v
        return carry

    lax.fori_loop(0, _ROWS_PER_BUF, _fill_row, 0)

    n_rows = out_hbm.shape[0]
    rows_per_worker = n_rows // _NUM_WORKERS
    base = wid * rows_per_worker
    copies = []
    for i in range(rows_per_worker // _ROWS_PER_BUF):
        dst = out_hbm.at[pl.ds(base + i * _ROWS_PER_BUF, _ROWS_PER_BUF), :]
        copies.append(pltpu.async_copy(buf, dst, sem))
    for c in copies:
        c.wait()


def kernel(hidden, obs, done):
    n_rows = obs.shape[1]
    mesh = plsc.VectorSubcoreMesh(core_axis_name="c", subcore_axis_name="s")
    fill = functools.partial(
        pl.kernel,
        mesh=mesh,
        out_type=jax.ShapeDtypeStruct((n_rows, _ACTION_DIM), jnp.float32),
        scratch_types=[
            pltpu.VMEM((_ROWS_PER_BUF, _ACTION_DIM), jnp.float32),
            pltpu.SemaphoreType.DMA,
        ],
    )(_fill_body)
    probs = fill()
    critic = jnp.asarray(0)
    return (hidden, probs, critic)


# speedup vs baseline: 1.4348x; 1.4348x over previous
import jax
import jax.numpy as jnp
from jax.experimental import pallas as pl
from jax.experimental.pallas import tpu as pltpu

_ACTION_DIM = 1024
_ACTION = (7, 42, 123)
_BUF_ROWS = 1024


def _probs_body(out_ref, buf, sem):
    col = jax.lax.broadcasted_iota(jnp.int32, (_BUF_ROWS, _ACTION_DIM), 1)
    mask = (col == _ACTION[0]) | (col == _ACTION[1]) | (col == _ACTION[2])
    buf[...] = mask.astype(jnp.float32)
    n = out_ref.shape[0] // _BUF_ROWS
    for i in range(n):
        dst = out_ref.at[pl.ds(i * _BUF_ROWS, _BUF_ROWS), :]
        pltpu.make_async_copy(buf, dst, sem).start()
    for i in range(n):
        dst = out_ref.at[pl.ds(i * _BUF_ROWS, _BUF_ROWS), :]
        pltpu.make_async_copy(buf, dst, sem).wait()


def kernel(hidden, obs, done):
    n_rows = obs.shape[1]
    probs = pl.pallas_call(
        _probs_body,
        out_specs=pl.BlockSpec(memory_space=pltpu.MemorySpace.HBM),
        out_shape=jax.ShapeDtypeStruct((n_rows, _ACTION_DIM), jnp.float32),
        scratch_shapes=[
            pltpu.VMEM((_BUF_ROWS, _ACTION_DIM), jnp.float32),
            pltpu.SemaphoreType.DMA,
        ],
    )()
    critic = jnp.asarray(0)
    return (hidden, probs, critic)
